# baseline (speedup 1.0000x reference)
import functools

import jax
import jax.numpy as jnp
import numpy as np
from jax import lax
from jax.experimental import pallas as pl
from jax.experimental.pallas import tpu as pltpu
from jax.experimental.pallas import tpu_sc as plsc

N_BINS = 15
N_ROWS = 65536
N_COLS = 1000
LOG2E = 1.4426950408889634

TC_ROWS = 49152
BLOCK_ROWS = 2048
TC_GRID = TC_ROWS // BLOCK_ROWS

SC_ROWS = N_ROWS - TC_ROWS
NW = 32
RPW = SC_ROWS // NW
CHUNK = 32
NCHUNK = RPW // CHUNK

_BOUNDS = np.linspace(0.0, 1.0, N_BINS + 1).astype(np.float32)
_LOWERS = np.full((1, 128), 2.0, np.float32)
_LOWERS[0, :N_BINS] = _BOUNDS[:-1]
_UPPERS = np.full((1, 128), 3.0, np.float32)
_UPPERS[0, :N_BINS] = _BOUNDS[1:]


def _tc_partials_kernel(bounds_ref, logits_ref, labels_ref, out_ref):
    x = logits_ref[...]
    m = jnp.max(x, axis=1, keepdims=True)
    c2 = bounds_ref[2:3, 0:1]
    s = jnp.sum(jnp.exp2((x - m) * c2), axis=1, keepdims=True)
    conf = 1.0 / s
    pred = jnp.argmax(x, axis=1).reshape(-1, 1)
    acc = (pred == labels_ref[...]).astype(jnp.float32)

    lowers = bounds_ref[0:1, :]
    uppers = bounds_ref[1:2, :]
    in_bin = ((conf > lowers) & (conf <= uppers)).astype(jnp.float32)
    out_ref[0, 0:1, :] = jnp.sum(in_bin, axis=0, keepdims=True)
    out_ref[0, 1:2, :] = jnp.sum(acc * in_bin, axis=0, keepdims=True)
    out_ref[0, 2:3, :] = jnp.sum(conf * in_bin, axis=0, keepdims=True)


def _combine_kernel(tc_ref, sc_ref, out_ref):
    p = tc_ref[...]
    q = sc_ref[...]
    count = jnp.sum(p[:, 0, :], axis=0, keepdims=True) + jnp.sum(
        q[:, 0, :], axis=0, keepdims=True)
    acc_sum = jnp.sum(p[:, 1, :], axis=0, keepdims=True) + jnp.sum(
        q[:, 1, :], axis=0, keepdims=True)
    conf_sum = jnp.sum(p[:, 2, :], axis=0, keepdims=True) + jnp.sum(
        q[:, 2, :], axis=0, keepdims=True)
    safe = jnp.maximum(count, 1.0)
    contrib = jnp.abs(conf_sum / safe - acc_sum / safe) * (count / N_ROWS)
    contrib = jnp.where(count > 0.0, contrib, 0.0)
    out_ref[...] = jnp.sum(contrib, axis=(0, 1), keepdims=True)


NROW16 = 16
NB4 = 15

@functools.partial(
    pl.kernel,
    out_type=jax.ShapeDtypeStruct((NW, 48), jnp.float32),
    mesh=plsc.VectorSubcoreMesh(core_axis_name="c", subcore_axis_name="s"),
    compiler_params=pltpu.CompilerParams(needs_layout_passes=False),
    scratch_types=[
        pltpu.VMEM((NROW16, N_COLS), jnp.float32),
        pltpu.VMEM((NROW16, N_COLS), jnp.float32),
        pltpu.VMEM((RPW,), jnp.int32),
        pltpu.VMEM((16,), jnp.float32),
        pltpu.VMEM((48,), jnp.float32),
        pltpu.SemaphoreType.DMA,
        pltpu.SemaphoreType.DMA,
    ],
)
def _sc_partials(logits_hbm, labels_hbm, scale_hbm, out_hbm,
                 buf0, buf1, lab_v, scal_v, stage, sem0, sem1):
    wid = lax.axis_index("s") * 2 + lax.axis_index("c")
    base = TC_ROWS + wid * RPW
    pltpu.sync_copy(scale_hbm, scal_v)
    pltpu.sync_copy(labels_hbm.at[pl.ds(base, RPW)], lab_v)
    sv = scal_v[...]
    lane = lax.iota(jnp.int32, 16)
    head8 = lane < 8
    neg_inf = jnp.full((16,), -jnp.inf, jnp.float32)
    zero16 = jnp.zeros((16,), jnp.float32)
    zero16i = jnp.zeros((16,), jnp.int32)
    nchunk = RPW // NROW16

    def start_copy(ci, buf, sem):
        return pltpu.async_copy(
            logits_hbm.at[pl.ds(base + ci * NROW16, NROW16)], buf, sem)

    def wait_copy(ci, buf, sem):
        pltpu.make_async_copy(
            logits_hbm.at[pl.ds(base + ci * NROW16, NROW16)], buf, sem).wait()

    def process(buf, ci, carry):
        def row_body(r, rcarry):
            cnt16, accs16, confs16 = rcarry

            def max_body(j, c):
                b0, i0, b1, i1, b2, i2, b3, i3 = c
                outs = []
                for k, (b, i) in enumerate(((b0, i0), (b1, i1),
                                            (b2, i2), (b3, i3))):
                    jj = j + k * NB4
                    v = buf[r, pl.ds(jj * 16, 16)]
                    gt = v > b
                    outs.append(jnp.where(gt, v, b))
                    outs.append(jnp.where(gt, jj * 16 + lane, i))
                return tuple(outs)

            c = lax.fori_loop(0, NB4, max_body,
                              (neg_inf, zero16i, neg_inf, zero16i,
                               neg_inf, zero16i, neg_inf, zero16i))
            b0, i0, b1, i1, b2, i2, b3, i3 = c
            for jj in (60, 61):
                v = buf[r, pl.ds(jj * 16, 16)]
                gt = v > b3
                b3 = jnp.where(gt, v, b3)
                i3 = jnp.where(gt, jj * 16 + lane, i3)
            vt = buf[r, pl.ds(N_COLS - 16, 16)]
            vt = jnp.where(head8, -jnp.inf, vt)
            gt = vt > b3
            b3 = jnp.where(gt, vt, b3)
            i3 = jnp.where(gt, N_COLS - 16 + lane, i3)
            gt = b1 > b0
            b0 = jnp.where(gt, b1, b0)
            i0 = jnp.where(gt, i1, i0)
            gt = b3 > b2
            b2 = jnp.where(gt, b3, b2)
            i2 = jnp.where(gt, i3, i2)
            gt = b2 > b0
            best = jnp.where(gt, b2, b0)
            bidx = jnp.where(gt, i2, i0)
            mv = jnp.full((16,), lax.reduce_max(best, (0,)), jnp.float32)
            predv = jnp.full(
                (16,),
                lax.reduce_min(
                    jnp.where(best == mv, bidx, jnp.int32(N_COLS)), (0,)),
                jnp.int32)

            def exp_body(j, c):
                s0, s1, s2, s3 = c
                outs = []
                for k, s in enumerate((s0, s1, s2, s3)):
                    v = buf[r, pl.ds((j + k * NB4) * 16, 16)]
                    outs.append(s + jnp.exp((v - mv) * sv))
                return tuple(outs)

            s0, s1, s2, s3 = lax.fori_loop(
                0, NB4, exp_body, (zero16, zero16, zero16, zero16))
            for jj in (60, 61):
                v = buf[r, pl.ds(jj * 16, 16)]
                s3 = s3 + jnp.exp((v - mv) * sv)
            s3 = s3 + jnp.where(head8, 0.0, jnp.exp((vt - mv) * sv))
            s16 = (s0 + s1) + (s2 + s3)
            confv = 1.0 / jnp.full((16,), lax.reduce_sum(s16, (0,)),
                                   jnp.float32)
            rhot = lane == r
            labv = lab_v[pl.ds(ci * NROW16, 16)]
            labrv = jnp.full(
                (16,),
                lax.reduce_max(jnp.where(rhot, labv, jnp.int32(-1)), (0,)),
                jnp.int32)
            accv = jnp.where(predv == labrv, 1.0, 0.0)
            y = confv * 15.0
            t = y.astype(jnp.int32)
            binv = t - jnp.where(t.astype(jnp.float32) == y, 1, 0)
            bhot = lane == binv
            cnt16 = cnt16 + jnp.where(bhot, 1.0, 0.0)
            accs16 = accs16 + jnp.where(bhot, accv, 0.0)
            confs16 = confs16 + jnp.where(bhot, confv, 0.0)
            return cnt16, accs16, confs16

        return lax.fori_loop(0, NROW16, row_body, carry)

    start_copy(0, buf0, sem0)

    def pair_body(h, carry):
        ci0 = h * 2
        wait_copy(ci0, buf0, sem0)
        start_copy(ci0 + 1, buf1, sem1)
        carry = process(buf0, ci0, carry)
        wait_copy(ci0 + 1, buf1, sem1)

        @pl.when(ci0 + 2 < nchunk)
        def _():
            start_copy(ci0 + 2, buf0, sem0)

        return process(buf1, ci0 + 1, carry)

    cnt16, accs16, confs16 = lax.fori_loop(
        0, nchunk // 2, pair_body, (zero16, zero16, zero16))
    stage[pl.ds(0, 16)] = cnt16
    stage[pl.ds(16, 16)] = accs16
    stage[pl.ds(32, 16)] = confs16
    pltpu.sync_copy(stage, out_hbm.at[wid])


@jax.jit
def _ece(logits, labels, bounds, scalevec):
    labels_i = labels.astype(jnp.int32)
    labels2 = labels_i.reshape(N_ROWS, 1)
    tc_parts = pl.pallas_call(
        _tc_partials_kernel,
        grid=(TC_GRID,),
        in_specs=[
            pl.BlockSpec((4, 128), lambda i: (0, 0)),
            pl.BlockSpec((BLOCK_ROWS, N_COLS), lambda i: (i, 0)),
            pl.BlockSpec((BLOCK_ROWS, 1), lambda i: (i, 0)),
        ],
        out_specs=pl.BlockSpec((1, 8, 128), lambda i: (i, 0, 0)),
        out_shape=jax.ShapeDtypeStruct((TC_GRID, 8, 128), jnp.float32),
        compiler_params=pltpu.CompilerParams(
            dimension_semantics=("parallel",)
        ),
    )(bounds, logits, labels2)
    sc_parts = _sc_partials(logits, labels_i, scalevec)
    sc_parts = jnp.pad(sc_parts.reshape(NW, 3, 16), ((0, 0), (0, 0), (0, 112)))
    out = pl.pallas_call(
        _combine_kernel,
        out_shape=jax.ShapeDtypeStruct((1, 1), jnp.float32),
    )(tc_parts, sc_parts)
    return out.reshape(1)


def kernel(logits, labels, t_opt):
    t = jnp.asarray(t_opt, jnp.float32)
    scale = jnp.where(t == 0.0, 1.0, 1.0 / t)
    bounds = np.zeros((4, 128), np.float32)
    bounds[0] = _LOWERS[0]
    bounds[1] = _UPPERS[0]
    b = jnp.asarray(bounds).at[2, 0].set(scale * LOG2E)
    scalevec = jnp.full((16,), scale, jnp.float32)
    return _ece(logits, labels, b, scalevec)


# speedup vs baseline: 1.0089x; 1.0089x over previous
import functools

import jax
import jax.numpy as jnp
import numpy as np
from jax import lax
from jax.experimental import pallas as pl
from jax.experimental.pallas import tpu as pltpu
from jax.experimental.pallas import tpu_sc as plsc

N_BINS = 15
N_ROWS = 65536
N_COLS = 1000
LOG2E = 1.4426950408889634

TC_ROWS = 49152
BLOCK_ROWS = 4096
TC_GRID = TC_ROWS // BLOCK_ROWS

SC_ROWS = N_ROWS - TC_ROWS
NW = 32
RPW = SC_ROWS // NW
CHUNK = 32
NCHUNK = RPW // CHUNK

_BOUNDS = np.linspace(0.0, 1.0, N_BINS + 1).astype(np.float32)
_LOWERS = np.full((1, 128), 2.0, np.float32)
_LOWERS[0, :N_BINS] = _BOUNDS[:-1]
_UPPERS = np.full((1, 128), 3.0, np.float32)
_UPPERS[0, :N_BINS] = _BOUNDS[1:]


def _tc_partials_kernel(bounds_ref, logits_ref, labels_ref, out_ref):
    x = logits_ref[...]
    m = jnp.max(x, axis=1, keepdims=True)
    c2 = bounds_ref[2:3, 0:1]
    s = jnp.sum(jnp.exp2((x - m) * c2), axis=1, keepdims=True)
    conf = 1.0 / s
    pred = jnp.argmax(x, axis=1).reshape(-1, 1)
    acc = (pred == labels_ref[...]).astype(jnp.float32)

    lowers = bounds_ref[0:1, :]
    uppers = bounds_ref[1:2, :]
    in_bin = ((conf > lowers) & (conf <= uppers)).astype(jnp.float32)
    out_ref[0, 0:1, :] = jnp.sum(in_bin, axis=0, keepdims=True)
    out_ref[0, 1:2, :] = jnp.sum(acc * in_bin, axis=0, keepdims=True)
    out_ref[0, 2:3, :] = jnp.sum(conf * in_bin, axis=0, keepdims=True)


def _combine_kernel(tc_ref, sc_ref, out_ref):
    p = tc_ref[...]
    q = sc_ref[...]
    count = jnp.sum(p[:, 0, :], axis=0, keepdims=True) + jnp.sum(
        q[:, 0, :], axis=0, keepdims=True)
    acc_sum = jnp.sum(p[:, 1, :], axis=0, keepdims=True) + jnp.sum(
        q[:, 1, :], axis=0, keepdims=True)
    conf_sum = jnp.sum(p[:, 2, :], axis=0, keepdims=True) + jnp.sum(
        q[:, 2, :], axis=0, keepdims=True)
    safe = jnp.maximum(count, 1.0)
    contrib = jnp.abs(conf_sum / safe - acc_sum / safe) * (count / N_ROWS)
    contrib = jnp.where(count > 0.0, contrib, 0.0)
    out_ref[...] = jnp.sum(contrib, axis=(0, 1), keepdims=True)


NROW16 = 16
NB4 = 15

@functools.partial(
    pl.kernel,
    out_type=jax.ShapeDtypeStruct((NW, 48), jnp.float32),
    mesh=plsc.VectorSubcoreMesh(core_axis_name="c", subcore_axis_name="s"),
    compiler_params=pltpu.CompilerParams(needs_layout_passes=False),
    scratch_types=[
        pltpu.VMEM((NROW16, N_COLS), jnp.float32),
        pltpu.VMEM((NROW16, N_COLS), jnp.float32),
        pltpu.VMEM((RPW,), jnp.int32),
        pltpu.VMEM((16,), jnp.float32),
        pltpu.VMEM((48,), jnp.float32),
        pltpu.SemaphoreType.DMA,
        pltpu.SemaphoreType.DMA,
    ],
)
def _sc_partials(logits_hbm, labels_hbm, scale_hbm, out_hbm,
                 buf0, buf1, lab_v, scal_v, stage, sem0, sem1):
    wid = lax.axis_index("s") * 2 + lax.axis_index("c")
    base = TC_ROWS + wid * RPW
    pltpu.sync_copy(scale_hbm, scal_v)
    pltpu.sync_copy(labels_hbm.at[pl.ds(base, RPW)], lab_v)
    sv = scal_v[...]
    lane = lax.iota(jnp.int32, 16)
    head8 = lane < 8
    neg_inf = jnp.full((16,), -jnp.inf, jnp.float32)
    zero16 = jnp.zeros((16,), jnp.float32)
    zero16i = jnp.zeros((16,), jnp.int32)
    nchunk = RPW // NROW16

    def start_copy(ci, buf, sem):
        return pltpu.async_copy(
            logits_hbm.at[pl.ds(base + ci * NROW16, NROW16)], buf, sem)

    def wait_copy(ci, buf, sem):
        pltpu.make_async_copy(
            logits_hbm.at[pl.ds(base + ci * NROW16, NROW16)], buf, sem).wait()

    def process(buf, ci, carry):
        def row_body(r, rcarry):
            cnt16, accs16, confs16 = rcarry

            def max_body(j, c):
                b0, i0, b1, i1, b2, i2, b3, i3 = c
                outs = []
                for k, (b, i) in enumerate(((b0, i0), (b1, i1),
                                            (b2, i2), (b3, i3))):
                    jj = j + k * NB4
                    v = buf[r, pl.ds(jj * 16, 16)]
                    gt = v > b
                    outs.append(jnp.where(gt, v, b))
                    outs.append(jnp.where(gt, jj * 16 + lane, i))
                return tuple(outs)

            c = lax.fori_loop(0, NB4, max_body,
                              (neg_inf, zero16i, neg_inf, zero16i,
                               neg_inf, zero16i, neg_inf, zero16i))
            b0, i0, b1, i1, b2, i2, b3, i3 = c
            for jj in (60, 61):
                v = buf[r, pl.ds(jj * 16, 16)]
                gt = v > b3
                b3 = jnp.where(gt, v, b3)
                i3 = jnp.where(gt, jj * 16 + lane, i3)
            vt = buf[r, pl.ds(N_COLS - 16, 16)]
            vt = jnp.where(head8, -jnp.inf, vt)
            gt = vt > b3
            b3 = jnp.where(gt, vt, b3)
            i3 = jnp.where(gt, N_COLS - 16 + lane, i3)
            gt = b1 > b0
            b0 = jnp.where(gt, b1, b0)
            i0 = jnp.where(gt, i1, i0)
            gt = b3 > b2
            b2 = jnp.where(gt, b3, b2)
            i2 = jnp.where(gt, i3, i2)
            gt = b2 > b0
            best = jnp.where(gt, b2, b0)
            bidx = jnp.where(gt, i2, i0)
            mv = jnp.full((16,), lax.reduce_max(best, (0,)), jnp.float32)
            predv = jnp.full(
                (16,),
                lax.reduce_min(
                    jnp.where(best == mv, bidx, jnp.int32(N_COLS)), (0,)),
                jnp.int32)

            def exp_body(j, c):
                s0, s1, s2, s3 = c
                outs = []
                for k, s in enumerate((s0, s1, s2, s3)):
                    v = buf[r, pl.ds((j + k * NB4) * 16, 16)]
                    outs.append(s + jnp.exp((v - mv) * sv))
                return tuple(outs)

            s0, s1, s2, s3 = lax.fori_loop(
                0, NB4, exp_body, (zero16, zero16, zero16, zero16))
            for jj in (60, 61):
                v = buf[r, pl.ds(jj * 16, 16)]
                s3 = s3 + jnp.exp((v - mv) * sv)
            s3 = s3 + jnp.where(head8, 0.0, jnp.exp((vt - mv) * sv))
            s16 = (s0 + s1) + (s2 + s3)
            confv = 1.0 / jnp.full((16,), lax.reduce_sum(s16, (0,)),
                                   jnp.float32)
            rhot = lane == r
            labv = lab_v[pl.ds(ci * NROW16, 16)]
            labrv = jnp.full(
                (16,),
                lax.reduce_max(jnp.where(rhot, labv, jnp.int32(-1)), (0,)),
                jnp.int32)
            accv = jnp.where(predv == labrv, 1.0, 0.0)
            y = confv * 15.0
            t = y.astype(jnp.int32)
            binv = t - jnp.where(t.astype(jnp.float32) == y, 1, 0)
            bhot = lane == binv
            cnt16 = cnt16 + jnp.where(bhot, 1.0, 0.0)
            accs16 = accs16 + jnp.where(bhot, accv, 0.0)
            confs16 = confs16 + jnp.where(bhot, confv, 0.0)
            return cnt16, accs16, confs16

        return lax.fori_loop(0, NROW16, row_body, carry)

    start_copy(0, buf0, sem0)

    def pair_body(h, carry):
        ci0 = h * 2
        wait_copy(ci0, buf0, sem0)
        start_copy(ci0 + 1, buf1, sem1)
        carry = process(buf0, ci0, carry)
        wait_copy(ci0 + 1, buf1, sem1)

        @pl.when(ci0 + 2 < nchunk)
        def _():
            start_copy(ci0 + 2, buf0, sem0)

        return process(buf1, ci0 + 1, carry)

    cnt16, accs16, confs16 = lax.fori_loop(
        0, nchunk // 2, pair_body, (zero16, zero16, zero16))
    stage[pl.ds(0, 16)] = cnt16
    stage[pl.ds(16, 16)] = accs16
    stage[pl.ds(32, 16)] = confs16
    pltpu.sync_copy(stage, out_hbm.at[wid])


@jax.jit
def _ece(logits, labels, bounds, scalevec):
    labels_i = labels.astype(jnp.int32)
    labels2 = labels_i.reshape(N_ROWS, 1)
    tc_parts = pl.pallas_call(
        _tc_partials_kernel,
        grid=(TC_GRID,),
        in_specs=[
            pl.BlockSpec((4, 128), lambda i: (0, 0)),
            pl.BlockSpec((BLOCK_ROWS, N_COLS), lambda i: (i, 0)),
            pl.BlockSpec((BLOCK_ROWS, 1), lambda i: (i, 0)),
        ],
        out_specs=pl.BlockSpec((1, 8, 128), lambda i: (i, 0, 0)),
        out_shape=jax.ShapeDtypeStruct((TC_GRID, 8, 128), jnp.float32),
        compiler_params=pltpu.CompilerParams(
            dimension_semantics=("parallel",)
        ),
    )(bounds, logits, labels2)
    sc_parts = _sc_partials(logits, labels_i, scalevec)
    sc_parts = jnp.pad(sc_parts.reshape(NW, 3, 16), ((0, 0), (0, 0), (0, 112)))
    out = pl.pallas_call(
        _combine_kernel,
        out_shape=jax.ShapeDtypeStruct((1, 1), jnp.float32),
    )(tc_parts, sc_parts)
    return out.reshape(1)


def kernel(logits, labels, t_opt):
    t = jnp.asarray(t_opt, jnp.float32)
    scale = jnp.where(t == 0.0, 1.0, 1.0 / t)
    bounds = np.zeros((4, 128), np.float32)
    bounds[0] = _LOWERS[0]
    bounds[1] = _UPPERS[0]
    b = jnp.asarray(bounds).at[2, 0].set(scale * LOG2E)
    scalevec = jnp.full((16,), scale, jnp.float32)
    return _ece(logits, labels, b, scalevec)
